# Initial kernel scaffold; baseline (speedup 1.0000x reference)
import jax
import jax.numpy as jnp
from jax.experimental import pallas as pl


def kernel(features, n_r, n_edge_index, poi_r, poi_edge_index, s_r, s_edge_index, d_r, d_edge_index, gcn_W0, gcn_b0, rel_W0, rel_b0, gcn_W1, gcn_b1, rel_W1, rel_b1, gcn_W2, gcn_b2, rel_W2, rel_b2, bn_g0, bn_b0, bn_g1, bn_b1, attn_Wqkv, attn_bqkv, attn_Wo, attn_bo, alpha_n, alpha_poi, alpha_s, alpha_d, fus_q, fus_W, fus_b):
    raise NotImplementedError("write your pallas kernel here")


# speedup vs baseline: 5.0434x; 5.0434x over previous
import functools

import jax
import jax.numpy as jnp
from jax import lax
from jax.experimental import pallas as pl
from jax.experimental.pallas import tpu as pltpu
from jax.experimental.pallas import tpu_sc as plsc

N = 10000
D = 256
E = 160000
NREL = 4
NSUB = 16
NCORE = 2
HALF = D // 2
CH = 128
NCHUNK = E // CH
_CPS = NCHUNK // NSUB
_CEXTRA = NCHUNK - _CPS * NSUB
RPW = (N // NSUB) // 8 * 8
RTAIL = N - RPW * NSUB

_MESH = plsc.VectorSubcoreMesh(core_axis_name="c", subcore_axis_name="s")


def _chunk_range(s):
    cnt = jnp.where(s < _CEXTRA, _CPS + 1, _CPS)
    start = jnp.where(s < _CEXTRA, (_CPS + 1) * s,
                      (_CPS + 1) * _CEXTRA + _CPS * (s - _CEXTRA))
    return start, cnt


def _deg_body(dst_hbm, ones_hbm, zeros_hbm, out_hbm, acc_sh, ones_v, idx_v):
    c = lax.axis_index("c")
    s = lax.axis_index("s")
    row0 = s * RPW
    pltpu.sync_copy(ones_hbm, ones_v)
    cstart, ccnt = _chunk_range(s)
    for rr in range(2):
        r = c * 2 + rr
        pltpu.sync_copy(zeros_hbm.at[pl.ds(row0, RPW)], acc_sh.at[pl.ds(row0, RPW)])

        @pl.when(s == NSUB - 1)
        def _():
            pltpu.sync_copy(zeros_hbm.at[pl.ds(N - RTAIL, RTAIL)],
                            acc_sh.at[pl.ds(N - RTAIL, RTAIL)])

        plsc.subcore_barrier()

        def body(j, _):
            base = r * E + (cstart + j) * CH
            pltpu.sync_copy(dst_hbm.at[pl.ds(base, CH)], idx_v)
            pltpu.sync_copy(ones_v, acc_sh.at[idx_v], add=True)
            return 0

        lax.fori_loop(0, ccnt, body, 0)
        plsc.subcore_barrier()
        pltpu.sync_copy(acc_sh.at[pl.ds(row0, RPW)], out_hbm.at[r, pl.ds(row0, RPW)])

        @pl.when(s == NSUB - 1)
        def _():
            pltpu.sync_copy(acc_sh.at[pl.ds(N - RTAIL, RTAIL)],
                            out_hbm.at[r, pl.ds(N - RTAIL, RTAIL)])

        plsc.subcore_barrier()


_deg_call = pl.kernel(
    _deg_body,
    out_type=jax.ShapeDtypeStruct((NREL, N, 16), jnp.float32),
    mesh=_MESH,
    scratch_types=[
        pltpu.VMEM_SHARED((N, 16), jnp.float32),
        pltpu.VMEM((CH, 16), jnp.float32),
        pltpu.VMEM((CH,), jnp.int32),
    ],
)


def _spmm_body(g_hbm, src_hbm, dst_hbm, zeros_hbm, out_hbm,
               acc_sh, rows_v, sidx_v, didx_v, sem):
    c = lax.axis_index("c")
    s = lax.axis_index("s")
    row0 = s * RPW
    cstart, ccnt = _chunk_range(s)
    for r in range(NREL):
        pltpu.sync_copy(zeros_hbm.at[pl.ds(row0, RPW)], acc_sh.at[pl.ds(row0, RPW)])

        @pl.when(s == NSUB - 1)
        def _():
            pltpu.sync_copy(zeros_hbm.at[pl.ds(N - RTAIL, RTAIL)],
                            acc_sh.at[pl.ds(N - RTAIL, RTAIL)])

        plsc.subcore_barrier()

        def body(j, _):
            base = r * E + (cstart + j) * CH
            pltpu.sync_copy(src_hbm.at[pl.ds(base, CH)], sidx_v)
            pltpu.sync_copy(dst_hbm.at[pl.ds(base, CH)], didx_v)
            pltpu.async_copy(g_hbm.at[c, r].at[sidx_v], rows_v, sem).wait()
            pltpu.sync_copy(rows_v, acc_sh.at[didx_v], add=True)
            return 0

        lax.fori_loop(0, ccnt, body, 0)
        plsc.subcore_barrier()
        pltpu.sync_copy(acc_sh.at[pl.ds(row0, RPW)], out_hbm.at[c, r, pl.ds(row0, RPW)])

        @pl.when(s == NSUB - 1)
        def _():
            pltpu.sync_copy(acc_sh.at[pl.ds(N - RTAIL, RTAIL)],
                            out_hbm.at[c, r, pl.ds(N - RTAIL, RTAIL)])

        plsc.subcore_barrier()


_spmm_call = pl.kernel(
    _spmm_body,
    out_type=jax.ShapeDtypeStruct((NCORE, NREL, N, HALF), jnp.float32),
    mesh=_MESH,
    scratch_types=[
        pltpu.VMEM_SHARED((N, HALF), jnp.float32),
        pltpu.VMEM((CH, HALF), jnp.float32),
        pltpu.VMEM((CH,), jnp.int32),
        pltpu.VMEM((CH,), jnp.int32),
        pltpu.SemaphoreType.DMA,
    ],
)


def kernel(features, n_r, n_edge_index, poi_r, poi_edge_index, s_r, s_edge_index,
           d_r, d_edge_index,
           gcn_W0, gcn_b0, rel_W0, rel_b0,
           gcn_W1, gcn_b1, rel_W1, rel_b1,
           gcn_W2, gcn_b2, rel_W2, rel_b2,
           bn_g0, bn_b0, bn_g1, bn_b1,
           attn_Wqkv, attn_bqkv, attn_Wo, attn_bo,
           alpha_n, alpha_poi, alpha_s, alpha_d,
           fus_q, fus_W, fus_b):
    src_all = jnp.stack([n_edge_index[0], poi_edge_index[0], s_edge_index[0],
                         d_edge_index[0]]).reshape(NREL * E)
    dst_all = jnp.stack([n_edge_index[1], poi_edge_index[1], s_edge_index[1],
                         d_edge_index[1]]).reshape(NREL * E)
    ones16 = jnp.ones((CH, 16), jnp.float32)
    zeros16 = jnp.zeros((N, 16), jnp.float32)
    zeros128 = jnp.zeros((N, HALF), jnp.float32)

    deg16 = _deg_call(dst_all, ones16, zeros16)
    deg = deg16[:, :, 0] + 1.0
    dinv = lax.rsqrt(deg)

    emb = jnp.broadcast_to(features, (NREL, N, D))
    rels = jnp.stack([n_r, poi_r, s_r, d_r])
    Ws = [gcn_W0, gcn_W1, gcn_W2]
    bs = [gcn_b0, gcn_b1, gcn_b2]
    rWs = [rel_W0, rel_W1, rel_W2]
    rbs = [rel_b0, rel_b1, rel_b2]
    bng = [bn_g0, bn_g1]
    bnb = [bn_b0, bn_b1]

    for i in range(3):
        x = emb * rels[:, None, :]
        h = jnp.einsum('rnd,ed->rne', x, Ws[i], preferred_element_type=jnp.float32)
        g = h * dinv[:, :, None]
        g2 = jnp.stack([g[..., :HALF], g[..., HALF:]])
        acc2 = _spmm_call(g2, src_all, dst_all, zeros128)
        acc = jnp.concatenate([acc2[0], acc2[1]], axis=-1)
        msg = (acc + g) * dinv[:, :, None] + bs[i]
        if i < 2:
            mu = msg.mean(axis=1, keepdims=True)
            var = msg.var(axis=1, keepdims=True)
            bn = (msg - mu) / jnp.sqrt(var + 1e-5) * bng[i] + bnb[i]
            emb = emb + jnp.where(bn >= 0, bn, 0.01 * bn)
            rels = rels @ rWs[i].T + rbs[i]
        else:
            emb = msg

    qkv = jnp.einsum('rnd,ed->rne', emb, attn_Wqkv, preferred_element_type=jnp.float32) + attn_bqkv
    q, k, v = jnp.split(qkv, 3, axis=-1)
    nh, hd = 4, D // 4

    def rs(t):
        return t.reshape(NREL, N * nh, hd).transpose(1, 0, 2)
    qh, kh, vh = rs(q), rs(k), rs(v)
    a = jax.nn.softmax(jnp.einsum('bld,bmd->blm', qh, kh) / jnp.sqrt(float(hd)), axis=-1)
    o = jnp.einsum('blm,bmd->bld', a, vh).transpose(1, 0, 2).reshape(NREL, N, D)
    fusion = jnp.einsum('rnd,ed->rne', o, attn_Wo, preferred_element_type=jnp.float32) + attn_bo

    alphas = jnp.stack([alpha_n, alpha_poi, alpha_s, alpha_d])
    f = fusion * alphas[:, None, None] + (1.0 - alphas[:, None, None]) * emb
    z = jnp.einsum('rnd,ed->rne', f, fus_W, preferred_element_type=jnp.float32) + fus_b
    z = jnp.where(z >= 0, z, 0.01 * z)
    wsum = jnp.mean(jnp.sum(z * fus_q, axis=2), axis=1)
    w = jax.nn.softmax(wsum)
    return jnp.einsum('r,rnd->nd', w, f)


# speedup vs baseline: 6.9403x; 1.3761x over previous
import functools

import numpy as np
import jax
import jax.numpy as jnp
from jax import lax
from jax.experimental import pallas as pl
from jax.experimental.pallas import tpu as pltpu
from jax.experimental.pallas import tpu_sc as plsc

N = 10000
D = 256
E = 160000
NREL = 4
NSUB = 16
NCORE = 2
HALF = D // 2
CH = 128
NCHUNK = E // CH
_CPS = NCHUNK // NSUB
_CEXTRA = NCHUNK - _CPS * NSUB
RPW = (N // NSUB) // 8 * 8
RTAIL = N - RPW * NSUB
BN_PRE = 2000
BN_MHA = 400
BN_FIN = 2000


def _chunk_range(s):
    cnt = jnp.where(s < _CEXTRA, _CPS + 1, _CPS)
    start = jnp.where(s < _CEXTRA, (_CPS + 1) * s,
                      (_CPS + 1) * _CEXTRA + _CPS * (s - _CEXTRA))
    return start, cnt


def _deg_body(dst_hbm, ones_hbm, zeros_hbm, out_hbm, acc_sh, ones_v, idx_v):
    c = lax.axis_index("c")
    s = lax.axis_index("s")
    row0 = s * RPW
    pltpu.sync_copy(ones_hbm, ones_v)
    cstart, ccnt = _chunk_range(s)
    for rr in range(2):
        r = c * 2 + rr
        pltpu.sync_copy(zeros_hbm.at[pl.ds(row0, RPW)], acc_sh.at[pl.ds(row0, RPW)])

        @pl.when(s == NSUB - 1)
        def _():
            pltpu.sync_copy(zeros_hbm.at[pl.ds(N - RTAIL, RTAIL)],
                            acc_sh.at[pl.ds(N - RTAIL, RTAIL)])

        plsc.subcore_barrier()

        def body(j, _):
            base = r * E + (cstart + j) * CH
            pltpu.sync_copy(dst_hbm.at[pl.ds(base, CH)], idx_v)
            pltpu.sync_copy(ones_v, acc_sh.at[idx_v], add=True)
            return 0

        lax.fori_loop(0, ccnt, body, 0)
        plsc.subcore_barrier()
        pltpu.sync_copy(acc_sh.at[pl.ds(row0, RPW)], out_hbm.at[r, pl.ds(row0, RPW)])

        @pl.when(s == NSUB - 1)
        def _():
            pltpu.sync_copy(acc_sh.at[pl.ds(N - RTAIL, RTAIL)],
                            out_hbm.at[r, pl.ds(N - RTAIL, RTAIL)])

        plsc.subcore_barrier()


@functools.cache
def _deg_call():
    return pl.kernel(
        _deg_body,
        out_type=jax.ShapeDtypeStruct((NREL, N, 16), jnp.float32),
        mesh=plsc.VectorSubcoreMesh(core_axis_name="c", subcore_axis_name="s"),
        scratch_types=[
            pltpu.VMEM_SHARED((N, 16), jnp.float32),
            pltpu.VMEM((CH, 16), jnp.float32),
            pltpu.VMEM((CH,), jnp.int32),
        ],
    )


def _spmm_body(g_hbm, src_hbm, dst_hbm, zeros_hbm, out_hbm,
               acc_sh, rows_v, sidx_v, didx_v, sem):
    c = lax.axis_index("c")
    s = lax.axis_index("s")
    row0 = s * RPW
    cstart, ccnt = _chunk_range(s)
    for r in range(NREL):
        pltpu.sync_copy(zeros_hbm.at[pl.ds(row0, RPW)], acc_sh.at[pl.ds(row0, RPW)])

        @pl.when(s == NSUB - 1)
        def _():
            pltpu.sync_copy(zeros_hbm.at[pl.ds(N - RTAIL, RTAIL)],
                            acc_sh.at[pl.ds(N - RTAIL, RTAIL)])

        plsc.subcore_barrier()

        def body(j, _):
            base = r * E + (cstart + j) * CH
            pltpu.sync_copy(src_hbm.at[pl.ds(base, CH)], sidx_v)
            pltpu.sync_copy(dst_hbm.at[pl.ds(base, CH)], didx_v)
            pltpu.async_copy(g_hbm.at[c, r].at[sidx_v], rows_v, sem).wait()
            pltpu.sync_copy(rows_v, acc_sh.at[didx_v], add=True)
            return 0

        lax.fori_loop(0, ccnt, body, 0)
        plsc.subcore_barrier()
        pltpu.sync_copy(acc_sh.at[pl.ds(row0, RPW)], out_hbm.at[c, r, pl.ds(row0, RPW)])

        @pl.when(s == NSUB - 1)
        def _():
            pltpu.sync_copy(acc_sh.at[pl.ds(N - RTAIL, RTAIL)],
                            out_hbm.at[c, r, pl.ds(N - RTAIL, RTAIL)])

        plsc.subcore_barrier()


@functools.cache
def _spmm_call():
    return pl.kernel(
        _spmm_body,
        out_type=jax.ShapeDtypeStruct((NCORE, NREL, N, HALF), jnp.float32),
        mesh=plsc.VectorSubcoreMesh(core_axis_name="c", subcore_axis_name="s"),
        scratch_types=[
            pltpu.VMEM_SHARED((N, HALF), jnp.float32),
            pltpu.VMEM((CH, HALF), jnp.float32),
            pltpu.VMEM((CH,), jnp.int32),
            pltpu.VMEM((CH,), jnp.int32),
            pltpu.SemaphoreType.DMA,
        ],
    )


def _leaky(x):
    return jnp.where(x >= 0, x, 0.01 * x)


def _dinv_from(deg_blk):
    return lax.rsqrt(deg_blk[:, 0:1] + 1.0)


def _sel_row(ref, r):
    row = ref[0:1]
    for rr in range(1, NREL):
        row = jnp.where(r == rr, ref[rr:rr + 1], row)
    return row


def _pre_body(emb_ref, rel_ref, wt_ref, deg_ref, out_ref, *, broadcast):
    x = (emb_ref[...] if broadcast else emb_ref[0]) * _sel_row(rel_ref, pl.program_id(0))
    h = jnp.dot(x, wt_ref[...], preferred_element_type=jnp.float32)
    g = h * _dinv_from(deg_ref[0])
    out_ref[0, 0] = g[:, :HALF]
    out_ref[1, 0] = g[:, HALF:]


def _make_pre(broadcast):
    nb = N // BN_PRE
    emb_spec = (pl.BlockSpec((BN_PRE, D), lambda r, b: (b, 0)) if broadcast
                else pl.BlockSpec((1, BN_PRE, D), lambda r, b: (r, b, 0)))
    return pl.pallas_call(
        functools.partial(_pre_body, broadcast=broadcast),
        grid=(NREL, nb),
        in_specs=[
            emb_spec,
            pl.BlockSpec((NREL, D), lambda r, b: (0, 0)),
            pl.BlockSpec((D, D), lambda r, b: (0, 0)),
            pl.BlockSpec((1, BN_PRE, 16), lambda r, b: (r, b, 0)),
        ],
        out_specs=pl.BlockSpec((2, 1, BN_PRE, HALF), lambda r, b: (0, r, b, 0)),
        out_shape=jax.ShapeDtypeStruct((2, NREL, N, HALF), jnp.float32),
    )


_pre_bcast = _make_pre(True)
_pre = _make_pre(False)


def _msg_of(acc_ref, g2_ref, deg_ref, b_ref):
    dinv = _dinv_from(deg_ref[0])
    acc = jnp.concatenate([acc_ref[0, 0], acc_ref[1, 0]], axis=-1)
    g = jnp.concatenate([g2_ref[0, 0], g2_ref[1, 0]], axis=-1)
    return (acc + g) * dinv + b_ref[...]


def _stats_body(acc_ref, g2_ref, deg_ref, b_ref, rels_ref, rwt_ref, rb_ref,
                stat_ref, rout_ref):
    nb = pl.program_id(1)
    msg = _msg_of(acc_ref, g2_ref, deg_ref, b_ref)
    s = jnp.sum(msg, axis=0, keepdims=True)
    sq = jnp.sum(msg * msg, axis=0, keepdims=True)

    @pl.when(nb == 0)
    def _():
        stat_ref[0, 0:1] = s
        stat_ref[0, 1:2] = sq

    @pl.when(nb > 0)
    def _():
        stat_ref[0, 0:1] = stat_ref[0, 0:1] + s
        stat_ref[0, 1:2] = stat_ref[0, 1:2] + sq

    @pl.when((pl.program_id(0) == 0) & (nb == 0))
    def _():
        rout_ref[...] = jnp.dot(rels_ref[...], rwt_ref[...],
                                preferred_element_type=jnp.float32) + rb_ref[...]


_stats_call = pl.pallas_call(
    _stats_body,
    grid=(NREL, N // BN_PRE),
    in_specs=[
        pl.BlockSpec((2, 1, BN_PRE, HALF), lambda r, b: (0, r, b, 0)),
        pl.BlockSpec((2, 1, BN_PRE, HALF), lambda r, b: (0, r, b, 0)),
        pl.BlockSpec((1, BN_PRE, 16), lambda r, b: (r, b, 0)),
        pl.BlockSpec((1, D), lambda r, b: (0, 0)),
        pl.BlockSpec((NREL, D), lambda r, b: (0, 0)),
        pl.BlockSpec((D, D), lambda r, b: (0, 0)),
        pl.BlockSpec((1, D), lambda r, b: (0, 0)),
    ],
    out_specs=[
        pl.BlockSpec((1, 2, D), lambda r, b: (r, 0, 0)),
        pl.BlockSpec((NREL, D), lambda r, b: (0, 0)),
    ],
    out_shape=[
        jax.ShapeDtypeStruct((NREL, 2, D), jnp.float32),
        jax.ShapeDtypeStruct((NREL, D), jnp.float32),
    ],
)


def _apply_body(acc_ref, g2_ref, deg_ref, b_ref, bng_ref, bnb_ref, emb_ref,
                stat_ref, out_ref, *, broadcast):
    msg = _msg_of(acc_ref, g2_ref, deg_ref, b_ref)
    mu = stat_ref[0, 0:1] / N
    var = stat_ref[0, 1:2] / N - mu * mu
    y = (msg - mu) * lax.rsqrt(var + 1e-5) * bng_ref[...] + bnb_ref[...]
    e = emb_ref[...] if broadcast else emb_ref[0]
    out_ref[0] = e + _leaky(y)


def _make_apply(broadcast):
    emb_spec = (pl.BlockSpec((BN_PRE, D), lambda r, b: (b, 0)) if broadcast
                else pl.BlockSpec((1, BN_PRE, D), lambda r, b: (r, b, 0)))
    return pl.pallas_call(
        functools.partial(_apply_body, broadcast=broadcast),
        grid=(NREL, N // BN_PRE),
        in_specs=[
            pl.BlockSpec((2, 1, BN_PRE, HALF), lambda r, b: (0, r, b, 0)),
            pl.BlockSpec((2, 1, BN_PRE, HALF), lambda r, b: (0, r, b, 0)),
            pl.BlockSpec((1, BN_PRE, 16), lambda r, b: (r, b, 0)),
            pl.BlockSpec((1, D), lambda r, b: (0, 0)),
            pl.BlockSpec((1, D), lambda r, b: (0, 0)),
            pl.BlockSpec((1, D), lambda r, b: (0, 0)),
            emb_spec,
            pl.BlockSpec((1, 2, D), lambda r, b: (r, 0, 0)),
        ],
        out_specs=pl.BlockSpec((1, BN_PRE, D), lambda r, b: (r, b, 0)),
        out_shape=jax.ShapeDtypeStruct((NREL, N, D), jnp.float32),
    )


_apply_bcast = _make_apply(True)
_apply = _make_apply(False)


def _mha_body(acc_ref, g2_ref, deg_ref, b_ref, wqkvt_ref, bqkv_ref, wot_ref,
              bo_ref, fuswt_ref, fusb_ref, fusq_ref, alpha_ref, p1_ref, p2_ref,
              f_ref, w_ref):
    nb = pl.program_id(0)
    es, qs, ks, vs = [], [], [], []
    for r in range(NREL):
        dinv = _dinv_from(deg_ref[r])
        acc = jnp.concatenate([acc_ref[0, r], acc_ref[1, r]], axis=-1)
        g = jnp.concatenate([g2_ref[0, r], g2_ref[1, r]], axis=-1)
        e = (acc + g) * dinv + b_ref[...]
        es.append(e)
        qkv = jnp.dot(e, wqkvt_ref[...], preferred_element_type=jnp.float32) + bqkv_ref[...]
        qs.append(qkv[:, :D])
        ks.append(qkv[:, D:2 * D])
        vs.append(qkv[:, 2 * D:])
    p1 = p1_ref[...]
    p2 = p2_ref[...]
    wparts = []
    for l1 in range(NREL):
        s = [jnp.dot(qs[l1] * ks[l2], p1, preferred_element_type=jnp.float32)
             for l2 in range(NREL)]
        m = jnp.maximum(jnp.maximum(s[0], s[1]), jnp.maximum(s[2], s[3]))
        ex = [jnp.exp(t - m) for t in s]
        den = ex[0] + ex[1] + ex[2] + ex[3]
        o = ex[0] / den @ p2 * vs[0]
        for l2 in range(1, NREL):
            o = o + (ex[l2] / den) @ p2 * vs[l2]
        fus = jnp.dot(o, wot_ref[...], preferred_element_type=jnp.float32) + bo_ref[...]
        al = alpha_ref[l1:l1 + 1, 0:1]
        f = fus * al + (1.0 - al) * es[l1]
        f_ref[l1] = f
        z = _leaky(jnp.dot(f, fuswt_ref[...], preferred_element_type=jnp.float32) + fusb_ref[...])
        wparts.append(jnp.sum(z * fusq_ref[...], axis=0, keepdims=True))
    w_ref[0] = jnp.concatenate(wparts, axis=0)


_mha_call = pl.pallas_call(
    _mha_body,
    grid=(N // BN_MHA,),
    in_specs=[
        pl.BlockSpec((2, NREL, BN_MHA, HALF), lambda b: (0, 0, b, 0)),
        pl.BlockSpec((2, NREL, BN_MHA, HALF), lambda b: (0, 0, b, 0)),
        pl.BlockSpec((NREL, BN_MHA, 16), lambda b: (0, b, 0)),
        pl.BlockSpec((1, D), lambda b: (0, 0)),
        pl.BlockSpec((D, 3 * D), lambda b: (0, 0)),
        pl.BlockSpec((1, 3 * D), lambda b: (0, 0)),
        pl.BlockSpec((D, D), lambda b: (0, 0)),
        pl.BlockSpec((1, D), lambda b: (0, 0)),
        pl.BlockSpec((D, D), lambda b: (0, 0)),
        pl.BlockSpec((1, D), lambda b: (0, 0)),
        pl.BlockSpec((1, D), lambda b: (0, 0)),
        pl.BlockSpec((NREL, 128), lambda b: (0, 0)),
        pl.BlockSpec((D, 8), lambda b: (0, 0)),
        pl.BlockSpec((8, D), lambda b: (0, 0)),
    ],
    out_specs=[
        pl.BlockSpec((NREL, BN_MHA, D), lambda b: (0, b, 0)),
        pl.BlockSpec((1, NREL, D), lambda b: (b, 0, 0)),
    ],
    out_shape=[
        jax.ShapeDtypeStruct((NREL, N, D), jnp.float32),
        jax.ShapeDtypeStruct((N // BN_MHA, NREL, D), jnp.float32),
    ],
)


def _final_body(f_ref, wsum_ref, onesb_ref, out_ref):
    ws = wsum_ref[0]
    for j in range(1, N // BN_MHA):
        ws = ws + wsum_ref[j]
    t = jnp.dot(ws, onesb_ref[...], preferred_element_type=jnp.float32)
    m = jnp.max(t, axis=0, keepdims=True)
    e = jnp.exp(t - m)
    w = e / jnp.sum(e, axis=0, keepdims=True)
    o = f_ref[0] * w[0:1, :]
    for r in range(1, NREL):
        o = o + f_ref[r] * w[r:r + 1, :]
    out_ref[...] = o


_final_call = pl.pallas_call(
    _final_body,
    grid=(N // BN_FIN,),
    in_specs=[
        pl.BlockSpec((NREL, BN_FIN, D), lambda b: (0, b, 0)),
        pl.BlockSpec((N // BN_MHA, NREL, D), lambda b: (0, 0, 0)),
        pl.BlockSpec((D, D), lambda b: (0, 0)),
    ],
    out_specs=pl.BlockSpec((BN_FIN, D), lambda b: (b, 0)),
    out_shape=jax.ShapeDtypeStruct((N, D), jnp.float32),
)


def _head_mats():
    p1 = np.zeros((D, 8), np.float32)
    p2 = np.zeros((8, D), np.float32)
    for dd in range(D):
        p1[dd, dd // 64] = 0.125
        p2[dd // 64, dd] = 1.0
    return jnp.asarray(p1), jnp.asarray(p2)


_P1, _P2 = _head_mats()


def kernel(features, n_r, n_edge_index, poi_r, poi_edge_index, s_r, s_edge_index,
           d_r, d_edge_index,
           gcn_W0, gcn_b0, rel_W0, rel_b0,
           gcn_W1, gcn_b1, rel_W1, rel_b1,
           gcn_W2, gcn_b2, rel_W2, rel_b2,
           bn_g0, bn_b0, bn_g1, bn_b1,
           attn_Wqkv, attn_bqkv, attn_Wo, attn_bo,
           alpha_n, alpha_poi, alpha_s, alpha_d,
           fus_q, fus_W, fus_b):
    src_all = jnp.stack([n_edge_index[0], poi_edge_index[0], s_edge_index[0],
                         d_edge_index[0]]).reshape(NREL * E)
    dst_all = jnp.stack([n_edge_index[1], poi_edge_index[1], s_edge_index[1],
                         d_edge_index[1]]).reshape(NREL * E)
    ones16 = jnp.ones((CH, 16), jnp.float32)
    zeros16 = jnp.zeros((N, 16), jnp.float32)
    zeros128 = jnp.zeros((N, HALF), jnp.float32)

    deg16 = _deg_call()(dst_all, ones16, zeros16)

    rels = jnp.stack([n_r, poi_r, s_r, d_r])
    Ws = [gcn_W0, gcn_W1, gcn_W2]
    bs = [gcn_b0, gcn_b1, gcn_b2]
    rWs = [rel_W0, rel_W1, rel_W2]
    rbs = [rel_b0, rel_b1, rel_b2]
    bng = [bn_g0, bn_g1]
    bnb = [bn_b0, bn_b1]

    emb = features
    for i in range(3):
        pre = _pre_bcast if i == 0 else _pre
        g2 = pre(emb, rels, Ws[i].T, deg16)
        acc2 = _spmm_call()(g2, src_all, dst_all, zeros128)
        if i < 2:
            stats, rels_new = _stats_call(acc2, g2, deg16, bs[i][None],
                                          rels, rWs[i].T, rbs[i][None])
            apply_ = _apply_bcast if i == 0 else _apply
            emb = apply_(acc2, g2, deg16, bs[i][None], bng[i][None], bnb[i][None],
                         emb, stats)
            rels = rels_new
        else:
            alphas = jnp.broadcast_to(
                jnp.stack([alpha_n, alpha_poi, alpha_s, alpha_d])[:, None], (4, 128))
            f, wsums = _mha_call(
                acc2, g2, deg16, bs[i][None], attn_Wqkv.T, attn_bqkv[None],
                attn_Wo.T, attn_bo[None], fus_W.T, fus_b[None],
                fus_q[None], alphas, _P1, _P2)
    onesb = jnp.full((D, D), 1.0 / N, jnp.float32)
    return _final_call(f, wsums, onesb)
